# Initial kernel scaffold; baseline (speedup 1.0000x reference)
import jax
import jax.numpy as jnp
from jax.experimental import pallas as pl


def kernel(scores, curr_scores, k):
    raise NotImplementedError("write your pallas kernel here")


# speedup vs baseline: 17.6986x; 17.6986x over previous
import functools

import jax
import jax.numpy as jnp
from jax import lax
from jax.experimental import pallas as pl
from jax.experimental.pallas import tpu as pltpu
from jax.experimental.pallas import tpu_sc as plsc

B = 32
N = 1_000_000
K2 = 32
CHUNK = 40_000
NCHUNKS = N // CHUNK
GROUP = 400
NGROUPS = CHUNK // GROUP
CAND = 64
BIGI = 2**31 - 1
BIGF = 3.0e38


def _scal_max(x):
    m = x[0]
    for i in range(1, 16):
        m = jnp.maximum(m, x[i])
    return m


def _scal_min(x):
    m = x[0]
    for i in range(1, 16):
        m = jnp.minimum(m, x[i])
    return m


def _merge_top32(cand_v, cand_i, top_v, top_i, lane):
    def one(j, carry):
        t0, t1, i0, i1 = carry
        acc = jnp.full((16,), -BIGF, jnp.float32)
        for i in range(CAND // 16):
            acc = jnp.maximum(acc, cand_v[pl.ds(i * 16, 16)])
        m = jnp.broadcast_to(_scal_max(acc), (16,))
        cacc = jnp.full((16,), BIGI, jnp.int32)
        for i in range(CAND // 16):
            v = cand_v[pl.ds(i * 16, 16)]
            c = cand_i[pl.ds(i * 16, 16)]
            cacc = jnp.minimum(cacc, jnp.where(v == m, c, BIGI))
        best = jnp.broadcast_to(_scal_min(cacc), (16,))
        for i in range(CAND // 16):
            v = cand_v[pl.ds(i * 16, 16)]
            c = cand_i[pl.ds(i * 16, 16)]
            cand_v[pl.ds(i * 16, 16)] = jnp.where(c == best, -BIGF, v)
        t0 = jnp.where(lane == j, m, t0)
        t1 = jnp.where(lane == (j - 16), m, t1)
        i0 = jnp.where(lane == j, best, i0)
        i1 = jnp.where(lane == (j - 16), best, i1)
        return t0, t1, i0, i1

    init = (jnp.full((16,), -BIGF, jnp.float32),
            jnp.full((16,), -BIGF, jnp.float32),
            jnp.full((16,), BIGI, jnp.int32), jnp.full((16,), BIGI, jnp.int32))
    t0, t1, i0, i1 = lax.fori_loop(0, K2, one, init)
    cand_v[pl.ds(0, 16)] = t0
    cand_v[pl.ds(16, 16)] = t1
    cand_i[pl.ds(0, 16)] = i0
    cand_i[pl.ds(16, 16)] = i1
    for i in range(2, CAND // 16):
        cand_v[pl.ds(i * 16, 16)] = jnp.full((16,), -BIGF, jnp.float32)
        cand_i[pl.ds(i * 16, 16)] = jnp.full((16,), BIGI, jnp.int32)
    top_v[pl.ds(0, 16)] = t0
    top_v[pl.ds(16, 16)] = t1
    top_i[pl.ds(0, 16)] = i0
    top_i[pl.ds(16, 16)] = i1
    return _scal_min(t1)


def _sc_row_topk(scores_flat):
    mesh = plsc.VectorSubcoreMesh(core_axis_name="c", subcore_axis_name="s")

    @functools.partial(
        pl.kernel,
        mesh=mesh,
        out_type=[
            jax.ShapeDtypeStruct((B * K2,), jnp.float32),
            jax.ShapeDtypeStruct((B * K2,), jnp.int32),
        ],
        scratch_types=[
            pltpu.VMEM((CHUNK,), jnp.float32),
            pltpu.VMEM((NGROUPS * 16,), jnp.float32),
            pltpu.VMEM((CAND,), jnp.float32),
            pltpu.VMEM((CAND,), jnp.int32),
            pltpu.VMEM((K2,), jnp.float32),
            pltpu.VMEM((K2,), jnp.int32),
        ],
    )
    def k(scores_hbm, out_v_hbm, out_i_hbm, buf, summ, cand_v, cand_i,
          top_v, top_i):
        wid = lax.axis_index("s") * 2 + lax.axis_index("c")
        lane = lax.iota(jnp.int32, 16)

        for i in range(CAND // 16):
            cand_v[pl.ds(i * 16, 16)] = jnp.full((16,), -BIGF, jnp.float32)
            cand_i[pl.ds(i * 16, 16)] = jnp.full((16,), BIGI, jnp.int32)

        def chunk_body(c, T):
            pltpu.sync_copy(scores_hbm.at[pl.ds(wid * N + c * CHUNK, CHUNK)], buf)
            v0 = buf[pl.ds(0, 16)]
            buf[pl.ds(0, 16)] = jnp.where(c * CHUNK + lane >= 2, v0, -BIGF)

            def group_body(g, _):
                off = g * GROUP
                acc = buf[pl.ds(off, 16)]
                for r in range(1, GROUP // 16):
                    acc = jnp.maximum(acc, buf[pl.ds(off + r * 16, 16)])
                summ[pl.ds(g * 16, 16)] = acc
                return 0

            lax.fori_loop(0, NGROUPS, group_body, 0)

            def summ_max(_, acc):
                return jnp.maximum(acc, summ[pl.ds(_ * 16, 16)])

            m0 = _scal_max(lax.fori_loop(
                0, NGROUPS, summ_max, jnp.full((16,), -BIGF, jnp.float32)))

            def round_body(j, st):
                cont, cnt, nextm, T1, prevm, prevcol = st

                def work(_, st2):
                    _cnt, _nextm, _prevm, _prevcol = st2
                    m16 = jnp.broadcast_to(_nextm, (16,))

                    def find_g(i, acc):
                        sv = summ[pl.ds(i * 16, 16)]
                        return jnp.minimum(acc, jnp.where(sv == m16, i, BIGI))

                    gstar = _scal_min(lax.fori_loop(
                        0, NGROUPS, find_g, jnp.full((16,), BIGI, jnp.int32)))
                    goff = gstar * GROUP
                    gbase = c * CHUNK + goff

                    lower = jnp.where(_nextm == _prevm, _prevcol, -1)
                    lower16 = jnp.broadcast_to(lower, (16,))

                    def find_col(r, acc):
                        v = buf[pl.ds(goff + r * 16, 16)]
                        col = lane + (gbase + r * 16)
                        return jnp.minimum(
                            acc,
                            jnp.where((v == m16) & (col > lower16), col, BIGI))

                    mincol = _scal_min(lax.fori_loop(
                        0, GROUP // 16, find_col,
                        jnp.full((16,), BIGI, jnp.int32)))
                    mincol16 = jnp.broadcast_to(mincol, (16,))

                    slot = 32 + (_cnt // 16) * 16
                    pos = _cnt % 16
                    cv = cand_v[pl.ds(slot, 16)]
                    ci = cand_i[pl.ds(slot, 16)]
                    cand_v[pl.ds(slot, 16)] = jnp.where(lane == pos, m16, cv)
                    cand_i[pl.ds(slot, 16)] = jnp.where(lane == pos, mincol16, ci)

                    def regroup(r, acc):
                        v = buf[pl.ds(goff + r * 16, 16)]
                        col = lane + (gbase + r * 16)
                        gone = (v > m16) | ((v == m16) & (col <= mincol16))
                        return jnp.maximum(acc, jnp.where(gone, -BIGF, v))

                    newsum = lax.fori_loop(0, GROUP // 16, regroup,
                                           jnp.full((16,), -BIGF, jnp.float32))
                    summ[pl.ds(gstar * 16, 16)] = newsum

                    nm = _scal_max(lax.fori_loop(
                        0, NGROUPS, summ_max,
                        jnp.full((16,), -BIGF, jnp.float32)))
                    return _cnt + 1, nm, _nextm, mincol

                cnt2, nextm2, prevm2, prevcol2 = lax.fori_loop(
                    0, cont, work, (cnt, nextm, prevm, prevcol))
                cont2 = ((nextm2 > T1) & (cnt2 < K2)).astype(jnp.int32)
                return cont2, cnt2, nextm2, T1, prevm2, prevcol2

            cont0 = (m0 > T).astype(jnp.int32)
            _, cntf, _, _, _, _ = lax.fori_loop(
                0, K2, round_body,
                (cont0, jnp.int32(0), m0, T, jnp.float32(BIGF), jnp.int32(-1)))

            def do_merge(_, T2):
                return _merge_top32(cand_v, cand_i, top_v, top_i, lane)

            return lax.fori_loop(0, (cntf > 0).astype(jnp.int32), do_merge, T)

        lax.fori_loop(0, NCHUNKS, chunk_body, jnp.float32(-BIGF))

        pltpu.sync_copy(top_v, out_v_hbm.at[pl.ds(wid * K2, K2)])
        pltpu.sync_copy(top_i, out_i_hbm.at[pl.ds(wid * K2, K2)])

    return k(scores_flat)


def _tc_merge(vals, cols, curr_scores):
    def body(v_ref, i_ref, c_ref, op_ref, ot_ref, oa_ref, ow_ref):
        vals = v_ref[...]
        cols = i_ref[...]
        curr = c_ref[...]
        valid = vals > 0.0
        lv = jnp.where(valid, jnp.log(vals) + curr, -jnp.inf)
        rows = lax.broadcasted_iota(jnp.int32, (B, K2), 0)
        flat = jnp.where(valid, rows * N + cols, BIGI)
        lane = lax.broadcasted_iota(jnp.int32, (1, K2), 1)
        res_v = jnp.zeros((1, K2), jnp.float32)
        res_f = jnp.zeros((1, K2), jnp.int32)
        for j in range(K2):
            m = jnp.max(lv)
            f = jnp.min(jnp.where(lv == m, flat, BIGI))
            res_v = jnp.where(lane == j, m, res_v)
            res_f = jnp.where(lane == j, f, res_f)
            lv = jnp.where(flat == f, -jnp.inf, lv)
        op_ref[...] = res_v
        ot_ref[...] = res_f
        oa_ref[...] = res_f // N
        ow_ref[...] = res_f % N

    return pl.pallas_call(
        body,
        out_shape=[
            jax.ShapeDtypeStruct((1, K2), jnp.float32),
            jax.ShapeDtypeStruct((1, K2), jnp.int32),
            jax.ShapeDtypeStruct((1, K2), jnp.int32),
            jax.ShapeDtypeStruct((1, K2), jnp.int32),
        ],
    )(vals, cols, curr_scores)


def kernel(scores, curr_scores, k):
    vals, cols = _sc_row_topk(scores.reshape(-1))
    maxp, top2k, anc, wrd = _tc_merge(vals.reshape(B, K2),
                                      cols.reshape(B, K2), curr_scores)
    return (maxp.reshape(-1), top2k.reshape(-1),
            anc.reshape(-1), wrd.reshape(-1))


# speedup vs baseline: 17.8439x; 1.0082x over previous
import functools

import jax
import jax.numpy as jnp
from jax import lax
from jax.experimental import pallas as pl
from jax.experimental.pallas import tpu as pltpu
from jax.experimental.pallas import tpu_sc as plsc

B = 32
N = 1_000_000
K2 = 32
CHUNK = 50_000
NCHUNKS = N // CHUNK
GROUP = 400
NGROUPS = CHUNK // GROUP
CAND = 64
BIGI = 2**31 - 1
BIGF = 3.0e38


def _scal_max(x):
    m = x[0]
    for i in range(1, 16):
        m = jnp.maximum(m, x[i])
    return m


def _scal_min(x):
    m = x[0]
    for i in range(1, 16):
        m = jnp.minimum(m, x[i])
    return m


def _merge_top32(cand_v, cand_i, top_v, top_i, lane):
    def one(j, carry):
        t0, t1, i0, i1 = carry
        acc = jnp.full((16,), -BIGF, jnp.float32)
        for i in range(CAND // 16):
            acc = jnp.maximum(acc, cand_v[pl.ds(i * 16, 16)])
        m = jnp.broadcast_to(_scal_max(acc), (16,))
        cacc = jnp.full((16,), BIGI, jnp.int32)
        for i in range(CAND // 16):
            v = cand_v[pl.ds(i * 16, 16)]
            c = cand_i[pl.ds(i * 16, 16)]
            cacc = jnp.minimum(cacc, jnp.where(v == m, c, BIGI))
        best = jnp.broadcast_to(_scal_min(cacc), (16,))
        for i in range(CAND // 16):
            v = cand_v[pl.ds(i * 16, 16)]
            c = cand_i[pl.ds(i * 16, 16)]
            cand_v[pl.ds(i * 16, 16)] = jnp.where(c == best, -BIGF, v)
        t0 = jnp.where(lane == j, m, t0)
        t1 = jnp.where(lane == (j - 16), m, t1)
        i0 = jnp.where(lane == j, best, i0)
        i1 = jnp.where(lane == (j - 16), best, i1)
        return t0, t1, i0, i1

    init = (jnp.full((16,), -BIGF, jnp.float32),
            jnp.full((16,), -BIGF, jnp.float32),
            jnp.full((16,), BIGI, jnp.int32), jnp.full((16,), BIGI, jnp.int32))
    t0, t1, i0, i1 = lax.fori_loop(0, K2, one, init)
    cand_v[pl.ds(0, 16)] = t0
    cand_v[pl.ds(16, 16)] = t1
    cand_i[pl.ds(0, 16)] = i0
    cand_i[pl.ds(16, 16)] = i1
    for i in range(2, CAND // 16):
        cand_v[pl.ds(i * 16, 16)] = jnp.full((16,), -BIGF, jnp.float32)
        cand_i[pl.ds(i * 16, 16)] = jnp.full((16,), BIGI, jnp.int32)
    top_v[pl.ds(0, 16)] = t0
    top_v[pl.ds(16, 16)] = t1
    top_i[pl.ds(0, 16)] = i0
    top_i[pl.ds(16, 16)] = i1
    return _scal_min(t1)


def _sc_row_topk(scores_flat):
    mesh = plsc.VectorSubcoreMesh(core_axis_name="c", subcore_axis_name="s")

    @functools.partial(
        pl.kernel,
        mesh=mesh,
        out_type=[
            jax.ShapeDtypeStruct((B * K2,), jnp.float32),
            jax.ShapeDtypeStruct((B * K2,), jnp.int32),
        ],
        scratch_types=[
            pltpu.VMEM((CHUNK,), jnp.float32),
            pltpu.VMEM((CHUNK,), jnp.float32),
            pltpu.VMEM((NGROUPS * 16,), jnp.float32),
            pltpu.VMEM((CAND,), jnp.float32),
            pltpu.VMEM((CAND,), jnp.int32),
            pltpu.VMEM((K2,), jnp.float32),
            pltpu.VMEM((K2,), jnp.int32),
            pltpu.SemaphoreType.DMA,
            pltpu.SemaphoreType.DMA,
        ],
    )
    def k(scores_hbm, out_v_hbm, out_i_hbm, buf0, buf1, summ, cand_v, cand_i,
          top_v, top_i, sem0, sem1):
        wid = lax.axis_index("s") * 2 + lax.axis_index("c")
        lane = lax.iota(jnp.int32, 16)

        for i in range(CAND // 16):
            cand_v[pl.ds(i * 16, 16)] = jnp.full((16,), -BIGF, jnp.float32)
            cand_i[pl.ds(i * 16, 16)] = jnp.full((16,), BIGI, jnp.int32)

        def src(c):
            return scores_hbm.at[pl.ds(wid * N + c * CHUNK, CHUNK)]

        def process(buf, c, T):
            v0 = buf[pl.ds(0, 16)]
            buf[pl.ds(0, 16)] = jnp.where(c * CHUNK + lane >= 2, v0, -BIGF)

            def group_body(g, _):
                off = g * GROUP
                vs = [buf[pl.ds(off + r * 16, 16)] for r in range(GROUP // 16)]
                while len(vs) > 1:
                    vs = [jnp.maximum(vs[i], vs[i + 1])
                          for i in range(0, len(vs) - 1, 2)] + (
                              [vs[-1]] if len(vs) % 2 else [])
                summ[pl.ds(g * 16, 16)] = vs[0]
                return 0

            lax.fori_loop(0, NGROUPS, group_body, 0)

            def summ_max(_, acc):
                return jnp.maximum(acc, summ[pl.ds(_ * 16, 16)])

            m0 = _scal_max(lax.fori_loop(
                0, NGROUPS, summ_max, jnp.full((16,), -BIGF, jnp.float32)))

            def round_body(j, st):
                cont, cnt, nextm, T1, prevm, prevcol = st

                def work(_, st2):
                    _cnt, _nextm, _prevm, _prevcol = st2
                    m16 = jnp.broadcast_to(_nextm, (16,))

                    def find_g(i, acc):
                        sv = summ[pl.ds(i * 16, 16)]
                        return jnp.minimum(acc, jnp.where(sv == m16, i, BIGI))

                    gstar = _scal_min(lax.fori_loop(
                        0, NGROUPS, find_g, jnp.full((16,), BIGI, jnp.int32)))
                    goff = gstar * GROUP
                    gbase = c * CHUNK + goff

                    lower = jnp.where(_nextm == _prevm, _prevcol, -1)
                    lower16 = jnp.broadcast_to(lower, (16,))

                    def find_col(r, acc):
                        v = buf[pl.ds(goff + r * 16, 16)]
                        col = lane + (gbase + r * 16)
                        return jnp.minimum(
                            acc,
                            jnp.where((v == m16) & (col > lower16), col, BIGI))

                    mincol = _scal_min(lax.fori_loop(
                        0, GROUP // 16, find_col,
                        jnp.full((16,), BIGI, jnp.int32)))
                    mincol16 = jnp.broadcast_to(mincol, (16,))

                    slot = 32 + (_cnt // 16) * 16
                    pos = _cnt % 16
                    cv = cand_v[pl.ds(slot, 16)]
                    ci = cand_i[pl.ds(slot, 16)]
                    cand_v[pl.ds(slot, 16)] = jnp.where(lane == pos, m16, cv)
                    cand_i[pl.ds(slot, 16)] = jnp.where(lane == pos, mincol16, ci)

                    def regroup(r, acc):
                        v = buf[pl.ds(goff + r * 16, 16)]
                        col = lane + (gbase + r * 16)
                        gone = (v > m16) | ((v == m16) & (col <= mincol16))
                        return jnp.maximum(acc, jnp.where(gone, -BIGF, v))

                    newsum = lax.fori_loop(0, GROUP // 16, regroup,
                                           jnp.full((16,), -BIGF, jnp.float32))
                    summ[pl.ds(gstar * 16, 16)] = newsum

                    nm = _scal_max(lax.fori_loop(
                        0, NGROUPS, summ_max,
                        jnp.full((16,), -BIGF, jnp.float32)))
                    return _cnt + 1, nm, _nextm, mincol

                cnt2, nextm2, prevm2, prevcol2 = lax.fori_loop(
                    0, cont, work, (cnt, nextm, prevm, prevcol))
                cont2 = ((nextm2 > T1) & (cnt2 < K2)).astype(jnp.int32)
                return cont2, cnt2, nextm2, T1, prevm2, prevcol2

            cont0 = (m0 > T).astype(jnp.int32)
            _, cntf, _, _, _, _ = lax.fori_loop(
                0, K2, round_body,
                (cont0, jnp.int32(0), m0, T, jnp.float32(BIGF), jnp.int32(-1)))

            def do_merge(_, T2):
                return _merge_top32(cand_v, cand_i, top_v, top_i, lane)

            return lax.fori_loop(0, (cntf > 0).astype(jnp.int32), do_merge, T)


        def step_body(sstep, T):
            c0 = sstep * 2
            pltpu.async_copy(src(c0 + 1), buf1, sem1)
            pltpu.make_async_copy(src(c0), buf0, sem0).wait()
            T = process(buf0, c0, T)
            nxt = jnp.minimum(c0 + 2, NCHUNKS - 1)
            pltpu.async_copy(src(nxt), buf0, sem0)
            pltpu.make_async_copy(src(c0 + 1), buf1, sem1).wait()
            return process(buf1, c0 + 1, T)

        pltpu.async_copy(src(0), buf0, sem0)
        lax.fori_loop(0, NCHUNKS // 2, step_body, jnp.float32(-BIGF))
        pltpu.make_async_copy(src(NCHUNKS - 1), buf0, sem0).wait()

        pltpu.sync_copy(top_v, out_v_hbm.at[pl.ds(wid * K2, K2)])
        pltpu.sync_copy(top_i, out_i_hbm.at[pl.ds(wid * K2, K2)])

    return k(scores_flat)


def _tc_merge(vals, cols, curr_scores):
    def body(v_ref, i_ref, c_ref, op_ref, ot_ref, oa_ref, ow_ref):
        vals = v_ref[...]
        cols = i_ref[...]
        curr = c_ref[...]
        valid = vals > 0.0
        lv = jnp.where(valid, jnp.log(vals) + curr, -jnp.inf)
        rows = lax.broadcasted_iota(jnp.int32, (B, K2), 0)
        flat = jnp.where(valid, rows * N + cols, BIGI)
        lane = lax.broadcasted_iota(jnp.int32, (1, K2), 1)
        res_v = jnp.zeros((1, K2), jnp.float32)
        res_f = jnp.zeros((1, K2), jnp.int32)
        for j in range(K2):
            m = jnp.max(lv)
            f = jnp.min(jnp.where(lv == m, flat, BIGI))
            res_v = jnp.where(lane == j, m, res_v)
            res_f = jnp.where(lane == j, f, res_f)
            lv = jnp.where(flat == f, -jnp.inf, lv)
        op_ref[...] = res_v
        ot_ref[...] = res_f
        oa_ref[...] = res_f // N
        ow_ref[...] = res_f % N

    return pl.pallas_call(
        body,
        out_shape=[
            jax.ShapeDtypeStruct((1, K2), jnp.float32),
            jax.ShapeDtypeStruct((1, K2), jnp.int32),
            jax.ShapeDtypeStruct((1, K2), jnp.int32),
            jax.ShapeDtypeStruct((1, K2), jnp.int32),
        ],
    )(vals, cols, curr_scores)


def kernel(scores, curr_scores, k):
    vals, cols = _sc_row_topk(scores.reshape(-1))
    maxp, top2k, anc, wrd = _tc_merge(vals.reshape(B, K2),
                                      cols.reshape(B, K2), curr_scores)
    return (maxp.reshape(-1), top2k.reshape(-1),
            anc.reshape(-1), wrd.reshape(-1))


# speedup vs baseline: 17.8718x; 1.0016x over previous
import functools

import jax
import jax.numpy as jnp
from jax import lax
from jax.experimental import pallas as pl
from jax.experimental.pallas import tpu as pltpu
from jax.experimental.pallas import tpu_sc as plsc

B = 32
N = 1_000_000
K2 = 32
CHUNK = 50_000
NCHUNKS = N // CHUNK
GROUP = 400
NGROUPS = CHUNK // GROUP
CAND = 64
BIGI = 2**31 - 1
BIGF = 3.0e38


def _scal_max(x):
    m = x[0]
    for i in range(1, 16):
        m = jnp.maximum(m, x[i])
    return m


def _scal_min(x):
    m = x[0]
    for i in range(1, 16):
        m = jnp.minimum(m, x[i])
    return m


def _merge_top32(cand_v, cand_i, top_v, top_i, lane):
    def one(j, carry):
        t0, t1, i0, i1 = carry
        acc = jnp.full((16,), -BIGF, jnp.float32)
        for i in range(CAND // 16):
            acc = jnp.maximum(acc, cand_v[pl.ds(i * 16, 16)])
        m = jnp.broadcast_to(_scal_max(acc), (16,))
        cacc = jnp.full((16,), BIGI, jnp.int32)
        for i in range(CAND // 16):
            v = cand_v[pl.ds(i * 16, 16)]
            c = cand_i[pl.ds(i * 16, 16)]
            cacc = jnp.minimum(cacc, jnp.where(v == m, c, BIGI))
        best = jnp.broadcast_to(_scal_min(cacc), (16,))
        for i in range(CAND // 16):
            v = cand_v[pl.ds(i * 16, 16)]
            c = cand_i[pl.ds(i * 16, 16)]
            cand_v[pl.ds(i * 16, 16)] = jnp.where(c == best, -BIGF, v)
        t0 = jnp.where(lane == j, m, t0)
        t1 = jnp.where(lane == (j - 16), m, t1)
        i0 = jnp.where(lane == j, best, i0)
        i1 = jnp.where(lane == (j - 16), best, i1)
        return t0, t1, i0, i1

    init = (jnp.full((16,), -BIGF, jnp.float32),
            jnp.full((16,), -BIGF, jnp.float32),
            jnp.full((16,), BIGI, jnp.int32), jnp.full((16,), BIGI, jnp.int32))
    t0, t1, i0, i1 = lax.fori_loop(0, K2, one, init)
    cand_v[pl.ds(0, 16)] = t0
    cand_v[pl.ds(16, 16)] = t1
    cand_i[pl.ds(0, 16)] = i0
    cand_i[pl.ds(16, 16)] = i1
    for i in range(2, CAND // 16):
        cand_v[pl.ds(i * 16, 16)] = jnp.full((16,), -BIGF, jnp.float32)
        cand_i[pl.ds(i * 16, 16)] = jnp.full((16,), BIGI, jnp.int32)
    top_v[pl.ds(0, 16)] = t0
    top_v[pl.ds(16, 16)] = t1
    top_i[pl.ds(0, 16)] = i0
    top_i[pl.ds(16, 16)] = i1
    return _scal_min(t1)


def _sc_row_topk(scores_flat):
    mesh = plsc.VectorSubcoreMesh(core_axis_name="c", subcore_axis_name="s")

    @functools.partial(
        pl.kernel,
        mesh=mesh,
        out_type=[
            jax.ShapeDtypeStruct((B * K2,), jnp.float32),
            jax.ShapeDtypeStruct((B * K2,), jnp.int32),
        ],
        scratch_types=[
            pltpu.VMEM((CHUNK,), jnp.float32),
            pltpu.VMEM((CHUNK,), jnp.float32),
            pltpu.VMEM((NGROUPS * 16,), jnp.float32),
            pltpu.VMEM((CAND,), jnp.float32),
            pltpu.VMEM((CAND,), jnp.int32),
            pltpu.VMEM((K2,), jnp.float32),
            pltpu.VMEM((K2,), jnp.int32),
            pltpu.SemaphoreType.DMA,
            pltpu.SemaphoreType.DMA,
        ],
    )
    def k(scores_hbm, out_v_hbm, out_i_hbm, buf0, buf1, summ, cand_v, cand_i,
          top_v, top_i, sem0, sem1):
        wid = lax.axis_index("s") * 2 + lax.axis_index("c")
        lane = lax.iota(jnp.int32, 16)

        for i in range(CAND // 16):
            cand_v[pl.ds(i * 16, 16)] = jnp.full((16,), -BIGF, jnp.float32)
            cand_i[pl.ds(i * 16, 16)] = jnp.full((16,), BIGI, jnp.int32)

        def src(c):
            return scores_hbm.at[pl.ds(wid * N + c * CHUNK, CHUNK)]

        def process(buf, c, T):
            v0 = buf[pl.ds(0, 16)]
            buf[pl.ds(0, 16)] = jnp.where(c * CHUNK + lane >= 2, v0, -BIGF)

            @plsc.parallel_loop(0, NGROUPS, unroll=4)
            def _(g):
                off = g * GROUP
                vs = [buf[pl.ds(off + r * 16, 16)] for r in range(GROUP // 16)]
                while len(vs) > 1:
                    vs = [jnp.maximum(vs[i], vs[i + 1])
                          for i in range(0, len(vs) - 1, 2)] + (
                              [vs[-1]] if len(vs) % 2 else [])
                summ[pl.ds(g * 16, 16)] = vs[0]

            def summ_max(_, acc):
                return jnp.maximum(acc, summ[pl.ds(_ * 16, 16)])

            m0 = _scal_max(lax.fori_loop(
                0, NGROUPS, summ_max, jnp.full((16,), -BIGF, jnp.float32)))

            def round_body(j, st):
                cont, cnt, nextm, T1, prevm, prevcol = st

                def work(_, st2):
                    _cnt, _nextm, _prevm, _prevcol = st2
                    m16 = jnp.broadcast_to(_nextm, (16,))

                    def find_g(i, acc):
                        sv = summ[pl.ds(i * 16, 16)]
                        return jnp.minimum(acc, jnp.where(sv == m16, i, BIGI))

                    gstar = _scal_min(lax.fori_loop(
                        0, NGROUPS, find_g, jnp.full((16,), BIGI, jnp.int32)))
                    goff = gstar * GROUP
                    gbase = c * CHUNK + goff

                    lower = jnp.where(_nextm == _prevm, _prevcol, -1)
                    lower16 = jnp.broadcast_to(lower, (16,))

                    def find_col(r, acc):
                        v = buf[pl.ds(goff + r * 16, 16)]
                        col = lane + (gbase + r * 16)
                        return jnp.minimum(
                            acc,
                            jnp.where((v == m16) & (col > lower16), col, BIGI))

                    mincol = _scal_min(lax.fori_loop(
                        0, GROUP // 16, find_col,
                        jnp.full((16,), BIGI, jnp.int32)))
                    mincol16 = jnp.broadcast_to(mincol, (16,))

                    slot = 32 + (_cnt // 16) * 16
                    pos = _cnt % 16
                    cv = cand_v[pl.ds(slot, 16)]
                    ci = cand_i[pl.ds(slot, 16)]
                    cand_v[pl.ds(slot, 16)] = jnp.where(lane == pos, m16, cv)
                    cand_i[pl.ds(slot, 16)] = jnp.where(lane == pos, mincol16, ci)

                    def regroup(r, acc):
                        v = buf[pl.ds(goff + r * 16, 16)]
                        col = lane + (gbase + r * 16)
                        gone = (v > m16) | ((v == m16) & (col <= mincol16))
                        return jnp.maximum(acc, jnp.where(gone, -BIGF, v))

                    newsum = lax.fori_loop(0, GROUP // 16, regroup,
                                           jnp.full((16,), -BIGF, jnp.float32))
                    summ[pl.ds(gstar * 16, 16)] = newsum

                    nm = _scal_max(lax.fori_loop(
                        0, NGROUPS, summ_max,
                        jnp.full((16,), -BIGF, jnp.float32)))
                    return _cnt + 1, nm, _nextm, mincol

                cnt2, nextm2, prevm2, prevcol2 = lax.fori_loop(
                    0, cont, work, (cnt, nextm, prevm, prevcol))
                cont2 = ((nextm2 > T1) & (cnt2 < K2)).astype(jnp.int32)
                return cont2, cnt2, nextm2, T1, prevm2, prevcol2

            cont0 = (m0 > T).astype(jnp.int32)
            _, cntf, _, _, _, _ = lax.fori_loop(
                0, K2, round_body,
                (cont0, jnp.int32(0), m0, T, jnp.float32(BIGF), jnp.int32(-1)))

            def do_merge(_, T2):
                return _merge_top32(cand_v, cand_i, top_v, top_i, lane)

            return lax.fori_loop(0, (cntf > 0).astype(jnp.int32), do_merge, T)


        def step_body(sstep, T):
            c0 = sstep * 2
            pltpu.async_copy(src(c0 + 1), buf1, sem1)
            pltpu.make_async_copy(src(c0), buf0, sem0).wait()
            T = process(buf0, c0, T)
            nxt = jnp.minimum(c0 + 2, NCHUNKS - 1)
            pltpu.async_copy(src(nxt), buf0, sem0)
            pltpu.make_async_copy(src(c0 + 1), buf1, sem1).wait()
            return process(buf1, c0 + 1, T)

        pltpu.async_copy(src(0), buf0, sem0)
        lax.fori_loop(0, NCHUNKS // 2, step_body, jnp.float32(-BIGF))
        pltpu.make_async_copy(src(NCHUNKS - 1), buf0, sem0).wait()

        pltpu.sync_copy(top_v, out_v_hbm.at[pl.ds(wid * K2, K2)])
        pltpu.sync_copy(top_i, out_i_hbm.at[pl.ds(wid * K2, K2)])

    return k(scores_flat)


def _tc_merge(vals, cols, curr_scores):
    def body(v_ref, i_ref, c_ref, op_ref, ot_ref, oa_ref, ow_ref):
        vals = v_ref[...]
        cols = i_ref[...]
        curr = c_ref[...]
        valid = vals > 0.0
        lv = jnp.where(valid, jnp.log(vals) + curr, -jnp.inf)
        rows = lax.broadcasted_iota(jnp.int32, (B, K2), 0)
        flat = jnp.where(valid, rows * N + cols, BIGI)
        lane = lax.broadcasted_iota(jnp.int32, (1, K2), 1)
        res_v = jnp.zeros((1, K2), jnp.float32)
        res_f = jnp.zeros((1, K2), jnp.int32)
        for j in range(K2):
            m = jnp.max(lv)
            f = jnp.min(jnp.where(lv == m, flat, BIGI))
            res_v = jnp.where(lane == j, m, res_v)
            res_f = jnp.where(lane == j, f, res_f)
            lv = jnp.where(flat == f, -jnp.inf, lv)
        op_ref[...] = res_v
        ot_ref[...] = res_f
        oa_ref[...] = res_f // N
        ow_ref[...] = res_f % N

    return pl.pallas_call(
        body,
        out_shape=[
            jax.ShapeDtypeStruct((1, K2), jnp.float32),
            jax.ShapeDtypeStruct((1, K2), jnp.int32),
            jax.ShapeDtypeStruct((1, K2), jnp.int32),
            jax.ShapeDtypeStruct((1, K2), jnp.int32),
        ],
    )(vals, cols, curr_scores)


def kernel(scores, curr_scores, k):
    vals, cols = _sc_row_topk(scores.reshape(-1))
    maxp, top2k, anc, wrd = _tc_merge(vals.reshape(B, K2),
                                      cols.reshape(B, K2), curr_scores)
    return (maxp.reshape(-1), top2k.reshape(-1),
            anc.reshape(-1), wrd.reshape(-1))
